# Initial kernel scaffold; baseline (speedup 1.0000x reference)
import jax
import jax.numpy as jnp
from jax.experimental import pallas as pl


def kernel(hidden_states, positional_params):
    raise NotImplementedError("write your pallas kernel here")


# speedup vs baseline: 7.2656x; 7.2656x over previous
import jax
import jax.numpy as jnp
from jax.experimental import pallas as pl

_SEQ = 512
_ROWS_PER_BLOCK = 8


def _copy_kernel(s_ref, out_ref):
    base = pl.program_id(0)
    off = (63 - base) * 8
    for r in range(_ROWS_PER_BLOCK):
        k = (7 - r) % 8
        out_ref[r] = s_ref[k, pl.ds(off, _SEQ), :]


def kernel(hidden_states, positional_params):
    seq = hidden_states.shape[1]
    hidden = positional_params.shape[1]
    shifted = jnp.stack(
        [jax.lax.dynamic_slice_in_dim(positional_params, k, 2 * seq - 8, axis=0)
         for k in range(8)]
    )
    grid = (seq // _ROWS_PER_BLOCK,)
    return pl.pallas_call(
        _copy_kernel,
        grid=grid,
        in_specs=[
            pl.BlockSpec(shifted.shape, lambda i: (0, 0, 0)),
        ],
        out_specs=pl.BlockSpec(
            (_ROWS_PER_BLOCK, seq, hidden), lambda i: (i, 0, 0)
        ),
        out_shape=jax.ShapeDtypeStruct((seq, seq, hidden), positional_params.dtype),
    )(shifted)
